# baseline (speedup 1.0000x reference)
import functools

import jax
import jax.numpy as jnp
from jax import lax
from jax.experimental import pallas as pl
from jax.experimental.pallas import tpu as pltpu
from jax.experimental.pallas import tpu_sc as plsc

_NW = 32
_L = 16


@functools.partial(jax.jit, static_argnums=(2,))
def _sc_gather_elements(table_flat, idx_flat, d):
    n_total = idx_flat.shape[0]
    n_per_w = n_total // _NW
    mesh = plsc.VectorSubcoreMesh(core_axis_name="c", subcore_axis_name="s")

    @functools.partial(
        pl.kernel,
        mesh=mesh,
        out_type=jax.ShapeDtypeStruct((n_total,), jnp.float32),
        scratch_types=[
            pltpu.VMEM((n_per_w,), jnp.int32),
            pltpu.VMEM((n_per_w,), jnp.float32),
            pltpu.SemaphoreType.DMA,
        ],
    )
    def k(table_hbm, idx_hbm, out_hbm, idx_v, out_v, sem):
        wid = lax.axis_index("s") * 2 + lax.axis_index("c")
        base = wid * n_per_w
        pltpu.sync_copy(idx_hbm.at[pl.ds(base, n_per_w)], idx_v)

        lanes = lax.iota(jnp.int32, _L)
        groups_per_row = d // _L

        def body(i, _):
            off = i * _L
            col = lax.rem(i, groups_per_row) * _L + lanes
            idx_v[pl.ds(off, _L)] = idx_v[pl.ds(off, _L)] * d + col
            return 0

        lax.fori_loop(0, n_per_w // _L, body, 0)

        pltpu.async_copy(table_hbm.at[idx_v], out_v, sem).wait()
        pltpu.sync_copy(out_v, out_hbm.at[pl.ds(base, n_per_w)])

    return k(table_flat, idx_flat)


def kernel(input_tensor, indices):
    r, d = input_tensor.shape
    b = indices.shape[0]
    idx_flat = indices.astype(jnp.int32).reshape(-1)
    out_flat = _sc_gather_elements(input_tensor.reshape(-1), idx_flat, d)
    return out_flat.reshape(b, d)
